# baseline (speedup 1.0000x reference)
import functools

import jax
import jax.numpy as jnp
from jax import lax
from jax.experimental import pallas as pl
from jax.experimental.pallas import tpu as pltpu
from jax.experimental.pallas import tpu_sc as plsc

D_MODEL = 64
DPAD = 128
SCALE = float(D_MODEL) ** 0.5

NC = 2
NS = 16
NW = NC * NS

S0 = 4096
S1 = 200
B = S0 * S1
G = 128
NG = B // G
GPW = NG // NW
RING = 8
DEPTH = 6
BLOCKS = GPW // RING


def _emb_body(x2d, table, out, idx_all, rows, gsem, ssem):
    wid = lax.axis_index("s") * NC + lax.axis_index("c")
    g0 = wid * GPW
    pltpu.sync_copy(x2d.at[pl.ds(g0, GPW)], idx_all)

    def gather_desc(g, s):
        return pltpu.make_async_copy(
            table.at[idx_all.at[g]], rows.at[pl.ds(s * G, G)], gsem.at[s]
        )

    def store_desc(g, s):
        return pltpu.make_async_copy(
            rows.at[pl.ds(s * G, G)],
            out.at[pl.ds((g0 + g) * G, G), pl.ds(0, D_MODEL)],
            ssem.at[s],
        )

    for g in range(DEPTH):
        gather_desc(g, g % RING).start()

    def blk_body(blk, carry):
        for b in range(RING):
            g = blk * RING + b
            h = g + DEPTH
            hs = (b + DEPTH) % RING

            @pl.when(h < GPW)
            def _fire():
                @pl.when(h >= RING)
                def _drain():
                    store_desc(h - RING, hs).wait()

                gather_desc(h, hs).start()

            gather_desc(g, b).wait()
            base = b * G

            def scale_row(i, c, _base=base):
                r = _base + i
                for j4 in range(D_MODEL // 16):
                    sl = (r, pl.ds(j4 * 16, 16))
                    rows[sl] = rows[sl] * SCALE
                return c

            lax.fori_loop(0, G, scale_row, 0, unroll=4)
            store_desc(g, b).start()
        return carry

    lax.fori_loop(0, BLOCKS, blk_body, 0)

    for b in range(RING):
        store_desc(GPW - RING + b, b).wait()


@functools.partial(jax.jit, static_argnames=())
def _emb_call(x2d, table):
    mesh = plsc.VectorSubcoreMesh(core_axis_name="c", subcore_axis_name="s")
    return pl.kernel(
        _emb_body,
        mesh=mesh,
        out_type=jax.ShapeDtypeStruct((B, DPAD), jnp.float32),
        scratch_types=[
            pltpu.VMEM((GPW, G), jnp.int32),
            pltpu.VMEM((RING * G, D_MODEL), jnp.float32),
            pltpu.SemaphoreType.DMA((RING,)),
            pltpu.SemaphoreType.DMA((RING,)),
        ],
        compiler_params=pltpu.CompilerParams(
            use_tc_tiling_on_sc=False, needs_layout_passes=False
        ),
    )(x2d, table)


def kernel(x, table):
    x2d = x.reshape(NG, G).astype(jnp.int32)
    out = _emb_call(x2d, table)
    return out[:, :D_MODEL].reshape(S0, S1, D_MODEL)
